# Initial kernel scaffold; baseline (speedup 1.0000x reference)
import jax
import jax.numpy as jnp
from jax.experimental import pallas as pl


def kernel(a_batch, a_coords, a_F, b_batch, b_coords, b_F):
    raise NotImplementedError("write your pallas kernel here")


# speedup vs baseline: 6.9269x; 6.9269x over previous
import functools

import jax
import jax.numpy as jnp
from jax import lax
from jax.experimental import pallas as pl
from jax.experimental.pallas import tpu as pltpu
from jax.experimental.pallas import tpu_sc as plsc

FULL_SCALE = 128.0
TOPK = 3
R = 0.5

BR = 256
BC = 1024
INF = float("inf")


def _topk_kernel(a_batch_ref, b_batch_ref, a_c_ref, b_c_ref,
                 w_ref, idx_ref, vals_s, idxs_s):
    j = pl.program_id(1)
    ncols = pl.num_programs(1)

    @pl.when(j == 0)
    def _init():
        vals_s[...] = jnp.full((TOPK, BR), INF, jnp.float32)
        idxs_s[...] = jnp.zeros((TOPK, BR), jnp.int32)

    a_b = a_batch_ref[0, :]
    b_b = b_batch_ref[0, :]
    active = jnp.logical_and(a_b[0] <= b_b[BC - 1], b_b[0] <= a_b[BR - 1])

    @pl.when(active)
    def _compute():
        a_c = (a_c_ref[...] // 16).astype(jnp.float32)
        b_c = (b_c_ref[...] // 16).astype(jnp.float32)
        d2 = jnp.zeros((BR, BC), jnp.float32)
        for d in range(3):
            diff = a_c[d, :][:, None] - b_c[d, :][None, :]
            d2 = d2 + diff * diff
        same = a_b[:, None] == b_b[None, :]
        d2 = jnp.where(same, d2, INF)
        col_ids = jax.lax.broadcasted_iota(jnp.int32, (BR, BC), 1) + j * BC

        t0 = vals_s[0, :]; t1 = vals_s[1, :]; t2 = vals_s[2, :]
        i0 = idxs_s[0, :]; i1 = idxs_s[1, :]; i2 = idxs_s[2, :]
        for _ in range(TOPK):
            c = jnp.min(d2, axis=1)
            ci = jnp.min(jnp.where(d2 == c[:, None], col_ids,
                                   jnp.int32(2**31 - 1)), axis=1)
            d2 = jnp.where(col_ids == ci[:, None], INF, d2)
            lt0 = c < t0; lt1 = c < t1; lt2 = c < t2
            t2 = jnp.where(lt1, t1, jnp.where(lt2, c, t2))
            i2 = jnp.where(lt1, i1, jnp.where(lt2, ci, i2))
            t1 = jnp.where(lt0, t0, jnp.where(lt1, c, t1))
            i1 = jnp.where(lt0, i0, jnp.where(lt1, ci, i1))
            t0 = jnp.where(lt0, c, t0)
            i0 = jnp.where(lt0, ci, i0)
        vals_s[0, :] = t0; vals_s[1, :] = t1; vals_s[2, :] = t2
        idxs_s[0, :] = i0; idxs_s[1, :] = i1; idxs_s[2, :] = i2

    @pl.when(j == ncols - 1)
    def _emit():
        v = vals_s[...]
        dist = jnp.sqrt(v) * jnp.float32(1.0 / FULL_SCALE)
        w_ref[...] = jnp.float32(R) - jnp.clip(dist, 0.0, jnp.float32(R))
        idx_ref[...] = idxs_s[...]


def _topk(a_batch, a_coords, b_batch, b_coords):
    Na = a_batch.shape[0]
    Nb = b_batch.shape[0]
    grid = (Na // BR, Nb // BC)
    w, idx = pl.pallas_call(
        _topk_kernel,
        grid=grid,
        in_specs=[
            pl.BlockSpec((1, BR), lambda i, j: (0, i)),
            pl.BlockSpec((1, BC), lambda i, j: (0, j)),
            pl.BlockSpec((3, BR), lambda i, j: (0, i)),
            pl.BlockSpec((3, BC), lambda i, j: (0, j)),
        ],
        out_specs=[
            pl.BlockSpec((TOPK, BR), lambda i, j: (0, i)),
            pl.BlockSpec((TOPK, BR), lambda i, j: (0, i)),
        ],
        out_shape=[
            jax.ShapeDtypeStruct((TOPK, Na), jnp.float32),
            jax.ShapeDtypeStruct((TOPK, Na), jnp.int32),
        ],
        scratch_shapes=[
            pltpu.VMEM((TOPK, BR), jnp.float32),
            pltpu.VMEM((TOPK, BR), jnp.int32),
        ],
        compiler_params=pltpu.CompilerParams(
            dimension_semantics=("arbitrary", "arbitrary")),
    )(a_batch.reshape(1, Na), b_batch.reshape(1, Nb),
      a_coords.T, b_coords.T)
    return w, idx


_CHUNK = 64


def _gather_sum(w, idx, b_F):
    Na = w.shape[1]
    D = b_F.shape[1]
    info = plsc.get_sparse_core_info()
    NW = info.num_cores * info.num_subcores
    rows_per_w = Na // NW
    nchunks = rows_per_w // _CHUNK
    mesh = plsc.VectorSubcoreMesh(core_axis_name="c", subcore_axis_name="s")

    @functools.partial(
        pl.kernel, mesh=mesh,
        out_type=jax.ShapeDtypeStruct((Na, D), jnp.float32),
        scratch_types=[
            pltpu.VMEM((_CHUNK,), jnp.int32),
            pltpu.VMEM((_CHUNK,), jnp.float32),
            pltpu.VMEM((_CHUNK, D), jnp.float32),
            pltpu.VMEM((_CHUNK, D), jnp.float32),
            pltpu.SemaphoreType.DMA,
        ],
    )
    def k(w_hbm, idx_hbm, bF_hbm, out_hbm, idx_v, w_v, rows_v, acc_v, sem):
        wid = lax.axis_index("s") * info.num_cores + lax.axis_index("c")
        base = wid * rows_per_w

        def chunk_body(c, _):
            cbase = base + c * _CHUNK
            for kk in range(TOPK):
                pltpu.sync_copy(idx_hbm.at[kk, pl.ds(cbase, _CHUNK)], idx_v)
                pltpu.sync_copy(w_hbm.at[kk, pl.ds(cbase, _CHUNK)], w_v)
                pltpu.async_copy(bF_hbm.at[idx_v], rows_v, sem).wait()

                def group_body(g, _):
                    w16 = w_v[pl.ds(g * 16, 16)]
                    for r in range(16):
                        wi = w16[r]
                        row = g * 16 + r
                        for jj in range(D // 16):
                            seg = rows_v[row, pl.ds(jj * 16, 16)] * wi
                            if kk == 0:
                                acc_v[row, pl.ds(jj * 16, 16)] = seg
                            else:
                                acc_v[row, pl.ds(jj * 16, 16)] += seg
                    return 0

                lax.fori_loop(0, _CHUNK // 16, group_body, 0)
            pltpu.sync_copy(acc_v, out_hbm.at[pl.ds(cbase, _CHUNK)])
            return 0

        lax.fori_loop(0, nchunks, chunk_body, 0)

    return k(w, idx, b_F)


def kernel(a_batch, a_coords, a_F, b_batch, b_coords, b_F):
    a_batch = a_batch.astype(jnp.int32)
    b_batch = b_batch.astype(jnp.int32)
    w, idx = _topk(a_batch, a_coords.astype(jnp.int32),
                   b_batch, b_coords.astype(jnp.int32))
    tmp = _gather_sum(w, idx, b_F)
    return jnp.concatenate([a_F, tmp], axis=-1)


# speedup vs baseline: 10.6897x; 1.5432x over previous
import functools

import jax
import jax.numpy as jnp
from jax import lax
from jax.experimental import pallas as pl
from jax.experimental.pallas import tpu as pltpu
from jax.experimental.pallas import tpu_sc as plsc

FULL_SCALE = 128.0
TOPK = 3
R = 0.5

BR = 256
BC = 2048
IMAX = 2**31 - 1


def _topk_kernel(a_batch_ref, b_batch_ref, a_c_ref, b_c_ref,
                 w_ref, idx_ref, keys_s):
    j = pl.program_id(1)
    ncols = pl.num_programs(1)

    @pl.when(j == 0)
    def _init():
        keys_s[...] = jnp.full((TOPK, BR), IMAX, jnp.int32)

    a_b = a_batch_ref[0, :]
    b_b = b_batch_ref[0, :]
    active = jnp.logical_and(a_b[0] <= b_b[BC - 1], b_b[0] <= a_b[BR - 1])

    @pl.when(active)
    def _compute():
        a_c = (a_c_ref[...] // 16).astype(jnp.float32)
        b_c = (b_c_ref[...] // 16).astype(jnp.float32)
        ab = jax.lax.dot_general(a_c, b_c, (((0,), (0,)), ((), ())),
                                 preferred_element_type=jnp.float32)
        a2 = jnp.sum(a_c * a_c, axis=0)
        b2 = jnp.sum(b_c * b_c, axis=0)
        d2 = a2[:, None] + (b2[None, :] - 2.0 * ab)
        col = jax.lax.broadcasted_iota(jnp.int32, (BR, BC), 1) + j * BC
        mismatch = (a_b[:, None] != b_b[None, :]).astype(jnp.int32) << 30
        p = d2.astype(jnp.int32) * 16384 + col + mismatch

        t0 = keys_s[0, :]; t1 = keys_s[1, :]; t2 = keys_s[2, :]
        for _ in range(TOPK):
            c = jnp.min(p, axis=1)
            p = jnp.where(p == c[:, None], IMAX, p)
            lt0 = c < t0; lt1 = c < t1; lt2 = c < t2
            t2 = jnp.where(lt1, t1, jnp.where(lt2, c, t2))
            t1 = jnp.where(lt0, t0, jnp.where(lt1, c, t1))
            t0 = jnp.where(lt0, c, t0)
        keys_s[0, :] = t0; keys_s[1, :] = t1; keys_s[2, :] = t2

    @pl.when(j == ncols - 1)
    def _emit():
        k = keys_s[...]
        idx_ref[...] = jnp.bitwise_and(k, 16383)
        d2 = jnp.right_shift(k, 14).astype(jnp.float32)
        dist = jnp.sqrt(d2) * jnp.float32(1.0 / FULL_SCALE)
        w_ref[...] = jnp.float32(R) - jnp.clip(dist, 0.0, jnp.float32(R))


def _topk(a_batch, a_coords, b_batch, b_coords):
    Na = a_batch.shape[0]
    Nb = b_batch.shape[0]
    grid = (Na // BR, Nb // BC)
    w, idx = pl.pallas_call(
        _topk_kernel,
        grid=grid,
        in_specs=[
            pl.BlockSpec((1, BR), lambda i, j: (0, i)),
            pl.BlockSpec((1, BC), lambda i, j: (0, j)),
            pl.BlockSpec((3, BR), lambda i, j: (0, i)),
            pl.BlockSpec((3, BC), lambda i, j: (0, j)),
        ],
        out_specs=[
            pl.BlockSpec((TOPK, BR), lambda i, j: (0, i)),
            pl.BlockSpec((TOPK, BR), lambda i, j: (0, i)),
        ],
        out_shape=[
            jax.ShapeDtypeStruct((TOPK, Na), jnp.float32),
            jax.ShapeDtypeStruct((TOPK, Na), jnp.int32),
        ],
        scratch_shapes=[
            pltpu.VMEM((TOPK, BR), jnp.int32),
        ],
        compiler_params=pltpu.CompilerParams(
            dimension_semantics=("arbitrary", "arbitrary")),
    )(a_batch.reshape(1, Na), b_batch.reshape(1, Nb),
      a_coords.T, b_coords.T)
    return w, idx


_CHUNK = 64


def _gather_sum(w, idx, b_F):
    Na = w.shape[1]
    D = b_F.shape[1]
    info = plsc.get_sparse_core_info()
    NW = info.num_cores * info.num_subcores
    rows_per_w = Na // NW
    nchunks = rows_per_w // _CHUNK
    mesh = plsc.VectorSubcoreMesh(core_axis_name="c", subcore_axis_name="s")

    @functools.partial(
        pl.kernel, mesh=mesh,
        out_type=jax.ShapeDtypeStruct((Na, D), jnp.float32),
        scratch_types=[
            pltpu.VMEM((_CHUNK,), jnp.int32),
            pltpu.VMEM((_CHUNK,), jnp.float32),
            pltpu.VMEM((_CHUNK, D), jnp.float32),
            pltpu.VMEM((_CHUNK, D), jnp.float32),
            pltpu.SemaphoreType.DMA,
        ],
    )
    def k(w_hbm, idx_hbm, bF_hbm, out_hbm, idx_v, w_v, rows_v, acc_v, sem):
        wid = lax.axis_index("s") * info.num_cores + lax.axis_index("c")
        base = wid * rows_per_w

        def chunk_body(c, _):
            cbase = base + c * _CHUNK
            for kk in range(TOPK):
                pltpu.sync_copy(idx_hbm.at[kk, pl.ds(cbase, _CHUNK)], idx_v)
                pltpu.sync_copy(w_hbm.at[kk, pl.ds(cbase, _CHUNK)], w_v)
                pltpu.async_copy(bF_hbm.at[idx_v], rows_v, sem).wait()

                def group_body(g, _):
                    w16 = w_v[pl.ds(g * 16, 16)]
                    for r in range(16):
                        wi = w16[r]
                        row = g * 16 + r
                        for jj in range(D // 16):
                            seg = rows_v[row, pl.ds(jj * 16, 16)] * wi
                            if kk == 0:
                                acc_v[row, pl.ds(jj * 16, 16)] = seg
                            else:
                                acc_v[row, pl.ds(jj * 16, 16)] += seg
                    return 0

                lax.fori_loop(0, _CHUNK // 16, group_body, 0)
            pltpu.sync_copy(acc_v, out_hbm.at[pl.ds(cbase, _CHUNK)])
            return 0

        lax.fori_loop(0, nchunks, chunk_body, 0)

    return k(w, idx, b_F)


def kernel(a_batch, a_coords, a_F, b_batch, b_coords, b_F):
    a_batch = a_batch.astype(jnp.int32)
    b_batch = b_batch.astype(jnp.int32)
    w, idx = _topk(a_batch, a_coords.astype(jnp.int32),
                   b_batch, b_coords.astype(jnp.int32))
    tmp = _gather_sum(w, idx, b_F)
    return jnp.concatenate([a_F, tmp], axis=-1)


# speedup vs baseline: 12.7511x; 1.1928x over previous
import functools

import jax
import jax.numpy as jnp
from jax import lax
from jax.experimental import pallas as pl
from jax.experimental.pallas import tpu as pltpu
from jax.experimental.pallas import tpu_sc as plsc

FULL_SCALE = 128.0
TOPK = 3
R = 0.5

BR = 256
BC = 2048
IMAX = 2**31 - 1


FMAX = float(jnp.finfo(jnp.float32).max)


def _merge33(a, b):
    x0 = jnp.minimum(a[0], b[0])
    u = jnp.maximum(a[0], b[0])
    v = jnp.minimum(a[1], b[1])
    x1 = jnp.minimum(u, v)
    x2 = jnp.minimum(jnp.maximum(u, v), jnp.minimum(a[2], b[2]))
    return (x0, x1, x2)


def _topk_kernel(a_batch_ref, b_batch_ref, a_c_ref, b_c_ref,
                 w_ref, idx_ref, keys_s):
    j = pl.program_id(1)
    ncols = pl.num_programs(1)

    @pl.when(j == 0)
    def _init():
        keys_s[...] = jnp.full((TOPK, BR, 128), FMAX, jnp.float32)

    a_b = a_batch_ref[0, :]
    b_b = b_batch_ref[0, :]
    active = jnp.logical_and(a_b[0] <= b_b[BC - 1], b_b[0] <= a_b[BR - 1])

    @pl.when(active)
    def _compute():
        a_c = (a_c_ref[...] // 16).astype(jnp.float32)
        b_c = (b_c_ref[...] // 16).astype(jnp.float32)
        ab2 = jax.lax.dot_general(a_c, b_c + b_c, (((0,), (0,)), ((), ())),
                                  preferred_element_type=jnp.float32)
        a2 = jnp.sum(a_c * a_c, axis=0)
        b2 = jnp.sum(b_c * b_c, axis=0)
        d2 = a2[:, None] + ((b2 + jnp.float32(512.0))[None, :] - ab2)
        ne = a_b[:, None] != b_b[None, :]
        d2 = jnp.where(ne, jnp.float32(66048.0), d2)
        col = jax.lax.broadcasted_iota(jnp.int32, (BR, BC), 1) + j * BC
        p = jnp.bitwise_or(jnp.left_shift(d2.astype(jnp.int32), 14), col)
        pf = jax.lax.bitcast_convert_type(p, jnp.float32)

        S = BC // 128
        sl = [pf[:, i * 128:(i + 1) * 128] for i in range(S)]
        lo = [jnp.minimum(sl[2 * i], sl[2 * i + 1]) for i in range(S // 2)]
        hi = [jnp.maximum(sl[2 * i], sl[2 * i + 1]) for i in range(S // 2)]

        def merge22(a0, a1, b0, b1):
            x0 = jnp.minimum(a0, b0)
            u = jnp.maximum(a0, b0)
            v = jnp.minimum(a1, b1)
            w = jnp.maximum(a1, b1)
            return (x0, jnp.minimum(u, v),
                    jnp.minimum(jnp.maximum(u, v), w))

        t3 = [merge22(lo[2 * i], hi[2 * i], lo[2 * i + 1], hi[2 * i + 1])
              for i in range(S // 4)]
        while len(t3) > 1:
            t3 = [_merge33(t3[2 * i], t3[2 * i + 1])
                  for i in range(len(t3) // 2)]

        run = (keys_s[0, :, :], keys_s[1, :, :], keys_s[2, :, :])
        m0, m1, m2 = _merge33(run, t3[0])
        keys_s[0, :, :] = m0
        keys_s[1, :, :] = m1
        keys_s[2, :, :] = m2

    @pl.when(j == ncols - 1)
    def _emit():
        r0 = keys_s[0, :, :]; r1 = keys_s[1, :, :]; r2 = keys_s[2, :, :]
        for k in range(TOPK):
            c = jnp.min(r0, axis=1)
            if k + 1 < TOPK:
                m = r0 == c[:, None]
                r0 = jnp.where(m, r1, r0)
                r1 = jnp.where(m, r2, r1)
                r2 = jnp.where(m, FMAX, r2)
            ci = jax.lax.bitcast_convert_type(c, jnp.int32)
            idx_ref[k, :] = jnp.bitwise_and(ci, 16383)
            d2 = (jnp.right_shift(ci, 14) - 512).astype(jnp.float32)
            dist = jnp.sqrt(d2) * jnp.float32(1.0 / FULL_SCALE)
            w_ref[k, :] = jnp.float32(R) - jnp.clip(dist, 0.0, jnp.float32(R))


def _topk(a_batch, a_coords, b_batch, b_coords):
    Na = a_batch.shape[0]
    Nb = b_batch.shape[0]
    grid = (Na // BR, Nb // BC)
    w, idx = pl.pallas_call(
        _topk_kernel,
        grid=grid,
        in_specs=[
            pl.BlockSpec((1, BR), lambda i, j: (0, i)),
            pl.BlockSpec((1, BC), lambda i, j: (0, j)),
            pl.BlockSpec((3, BR), lambda i, j: (0, i)),
            pl.BlockSpec((3, BC), lambda i, j: (0, j)),
        ],
        out_specs=[
            pl.BlockSpec((TOPK, BR), lambda i, j: (0, i)),
            pl.BlockSpec((TOPK, BR), lambda i, j: (0, i)),
        ],
        out_shape=[
            jax.ShapeDtypeStruct((TOPK, Na), jnp.float32),
            jax.ShapeDtypeStruct((TOPK, Na), jnp.int32),
        ],
        scratch_shapes=[
            pltpu.VMEM((TOPK, BR, 128), jnp.float32),
        ],
        compiler_params=pltpu.CompilerParams(
            dimension_semantics=("arbitrary", "arbitrary")),
    )(a_batch.reshape(1, Na), b_batch.reshape(1, Nb),
      a_coords.T, b_coords.T)
    return w, idx


_CHUNK = 64


def _gather_sum(w, idx, b_F):
    Na = w.shape[1]
    D = b_F.shape[1]
    info = plsc.get_sparse_core_info()
    NW = info.num_cores * info.num_subcores
    rows_per_w = Na // NW
    nchunks = rows_per_w // _CHUNK
    mesh = plsc.VectorSubcoreMesh(core_axis_name="c", subcore_axis_name="s")

    @functools.partial(
        pl.kernel, mesh=mesh,
        out_type=jax.ShapeDtypeStruct((Na, D), jnp.float32),
        scratch_types=[
            pltpu.VMEM((_CHUNK,), jnp.int32),
            pltpu.VMEM((_CHUNK,), jnp.float32),
            pltpu.VMEM((_CHUNK, D), jnp.float32),
            pltpu.VMEM((_CHUNK, D), jnp.float32),
            pltpu.SemaphoreType.DMA,
        ],
    )
    def k(w_hbm, idx_hbm, bF_hbm, out_hbm, idx_v, w_v, rows_v, acc_v, sem):
        wid = lax.axis_index("s") * info.num_cores + lax.axis_index("c")
        base = wid * rows_per_w

        def chunk_body(c, _):
            cbase = base + c * _CHUNK
            for kk in range(TOPK):
                pltpu.sync_copy(idx_hbm.at[kk, pl.ds(cbase, _CHUNK)], idx_v)
                pltpu.sync_copy(w_hbm.at[kk, pl.ds(cbase, _CHUNK)], w_v)
                pltpu.async_copy(bF_hbm.at[idx_v], rows_v, sem).wait()

                def group_body(g, _):
                    w16 = w_v[pl.ds(g * 16, 16)]
                    for r in range(16):
                        wi = w16[r]
                        row = g * 16 + r
                        for jj in range(D // 16):
                            seg = rows_v[row, pl.ds(jj * 16, 16)] * wi
                            if kk == 0:
                                acc_v[row, pl.ds(jj * 16, 16)] = seg
                            else:
                                acc_v[row, pl.ds(jj * 16, 16)] += seg
                    return 0

                lax.fori_loop(0, _CHUNK // 16, group_body, 0)
            pltpu.sync_copy(acc_v, out_hbm.at[pl.ds(cbase, _CHUNK)])
            return 0

        lax.fori_loop(0, nchunks, chunk_body, 0)

    return k(w, idx, b_F)


def kernel(a_batch, a_coords, a_F, b_batch, b_coords, b_F):
    a_batch = a_batch.astype(jnp.int32)
    b_batch = b_batch.astype(jnp.int32)
    w, idx = _topk(a_batch, a_coords.astype(jnp.int32),
                   b_batch, b_coords.astype(jnp.int32))
    tmp = _gather_sum(w, idx, b_F)
    return jnp.concatenate([a_F, tmp], axis=-1)


# speedup vs baseline: 12.9543x; 1.0159x over previous
import functools

import jax
import jax.numpy as jnp
from jax import lax
from jax.experimental import pallas as pl
from jax.experimental.pallas import tpu as pltpu
from jax.experimental.pallas import tpu_sc as plsc

FULL_SCALE = 128.0
TOPK = 3
R = 0.5

BR = 256
BC = 2048
IMAX = 2**31 - 1


FMAX = float(jnp.finfo(jnp.float32).max)


def _merge33(a, b):
    x0 = jnp.minimum(a[0], b[0])
    u = jnp.maximum(a[0], b[0])
    v = jnp.minimum(a[1], b[1])
    x1 = jnp.minimum(u, v)
    x2 = jnp.minimum(jnp.maximum(u, v), jnp.minimum(a[2], b[2]))
    return (x0, x1, x2)


def _topk_kernel(a_batch_ref, b_batch_ref, a_c_ref, b_c_ref,
                 w_ref, idx_ref, keys_s):
    j = pl.program_id(1)
    ncols = pl.num_programs(1)

    @pl.when(j == 0)
    def _init():
        keys_s[...] = jnp.full((TOPK, BR, 128), FMAX, jnp.float32)

    a_b = a_batch_ref[0, :]
    b_b = b_batch_ref[0, :]
    active = jnp.logical_and(a_b[0] <= b_b[BC - 1], b_b[0] <= a_b[BR - 1])

    @pl.when(active)
    def _compute():
        a_c = (a_c_ref[...] // 16).astype(jnp.float32)
        b_c = (b_c_ref[...] // 16).astype(jnp.float32)
        ab2 = jax.lax.dot_general(a_c, b_c + b_c, (((0,), (0,)), ((), ())),
                                  preferred_element_type=jnp.float32)
        a2 = jnp.sum(a_c * a_c, axis=0)
        b2 = jnp.sum(b_c * b_c, axis=0)
        d2 = a2[:, None] + ((b2 + jnp.float32(512.0))[None, :] - ab2)
        ne = a_b[:, None] != b_b[None, :]
        d2 = jnp.where(ne, jnp.float32(66048.0), d2)
        col = jax.lax.broadcasted_iota(jnp.int32, (BR, BC), 1) + j * BC
        p = jnp.bitwise_or(jnp.left_shift(d2.astype(jnp.int32), 14), col)
        pf = jax.lax.bitcast_convert_type(p, jnp.float32)

        S = BC // 128
        sl = [pf[:, i * 128:(i + 1) * 128] for i in range(S)]
        lo = [jnp.minimum(sl[2 * i], sl[2 * i + 1]) for i in range(S // 2)]
        hi = [jnp.maximum(sl[2 * i], sl[2 * i + 1]) for i in range(S // 2)]

        def merge22(a0, a1, b0, b1):
            x0 = jnp.minimum(a0, b0)
            u = jnp.maximum(a0, b0)
            v = jnp.minimum(a1, b1)
            w = jnp.maximum(a1, b1)
            return (x0, jnp.minimum(u, v),
                    jnp.minimum(jnp.maximum(u, v), w))

        t3 = [merge22(lo[2 * i], hi[2 * i], lo[2 * i + 1], hi[2 * i + 1])
              for i in range(S // 4)]
        while len(t3) > 1:
            t3 = [_merge33(t3[2 * i], t3[2 * i + 1])
                  for i in range(len(t3) // 2)]

        run = (keys_s[0, :, :], keys_s[1, :, :], keys_s[2, :, :])
        m0, m1, m2 = _merge33(run, t3[0])
        keys_s[0, :, :] = m0
        keys_s[1, :, :] = m1
        keys_s[2, :, :] = m2

    @pl.when(j == ncols - 1)
    def _emit():
        r0 = keys_s[0, :, :]; r1 = keys_s[1, :, :]; r2 = keys_s[2, :, :]
        for k in range(TOPK):
            c = jnp.min(r0, axis=1)
            if k + 1 < TOPK:
                m = r0 == c[:, None]
                r0 = jnp.where(m, r1, r0)
                r1 = jnp.where(m, r2, r1)
                r2 = jnp.where(m, FMAX, r2)
            ci = jax.lax.bitcast_convert_type(c, jnp.int32)
            idx_ref[k, :] = jnp.bitwise_and(ci, 16383)
            d2 = (jnp.right_shift(ci, 14) - 512).astype(jnp.float32)
            dist = jnp.sqrt(d2) * jnp.float32(1.0 / FULL_SCALE)
            w_ref[k, :] = jnp.float32(R) - jnp.clip(dist, 0.0, jnp.float32(R))


def _topk(a_batch, a_coords, b_batch, b_coords):
    Na = a_batch.shape[0]
    Nb = b_batch.shape[0]
    grid = (Na // BR, Nb // BC)
    w, idx = pl.pallas_call(
        _topk_kernel,
        grid=grid,
        in_specs=[
            pl.BlockSpec((1, BR), lambda i, j: (0, i)),
            pl.BlockSpec((1, BC), lambda i, j: (0, j)),
            pl.BlockSpec((3, BR), lambda i, j: (0, i)),
            pl.BlockSpec((3, BC), lambda i, j: (0, j)),
        ],
        out_specs=[
            pl.BlockSpec((TOPK, BR), lambda i, j: (0, i)),
            pl.BlockSpec((TOPK, BR), lambda i, j: (0, i)),
        ],
        out_shape=[
            jax.ShapeDtypeStruct((TOPK, Na), jnp.float32),
            jax.ShapeDtypeStruct((TOPK, Na), jnp.int32),
        ],
        scratch_shapes=[
            pltpu.VMEM((TOPK, BR, 128), jnp.float32),
        ],
        compiler_params=pltpu.CompilerParams(
            dimension_semantics=("arbitrary", "arbitrary")),
    )(a_batch.reshape(1, Na), b_batch.reshape(1, Nb),
      a_coords.T, b_coords.T)
    return w, idx


_CHUNK = 64


def _gather_sum(w, idx, b_F):
    Na = w.shape[1]
    D = b_F.shape[1]
    info = plsc.get_sparse_core_info()
    NW = info.num_cores * info.num_subcores
    rows_per_w = Na // NW
    nchunks = rows_per_w // _CHUNK
    mesh = plsc.VectorSubcoreMesh(core_axis_name="c", subcore_axis_name="s")

    @functools.partial(
        pl.kernel, mesh=mesh,
        out_type=jax.ShapeDtypeStruct((Na, D), jnp.float32),
        scratch_types=[
            pltpu.VMEM((TOPK, _CHUNK), jnp.int32),
            pltpu.VMEM((TOPK, _CHUNK), jnp.float32),
            pltpu.VMEM((_CHUNK, D), jnp.float32),
            pltpu.VMEM((_CHUNK, D), jnp.float32),
            pltpu.VMEM((_CHUNK, D), jnp.float32),
            pltpu.SemaphoreType.DMA,
        ],
    )
    def k(w_hbm, idx_hbm, bF_hbm, out_hbm, idx_v, w_v, r0_v, r1_v, r2_v, sem):
        wid = lax.axis_index("s") * info.num_cores + lax.axis_index("c")
        base = wid * rows_per_w
        bufs = (r0_v, r1_v, r2_v)

        def chunk_body(c, _):
            cbase = base + c * _CHUNK
            for kk in range(TOPK):
                pltpu.sync_copy(idx_hbm.at[kk, pl.ds(cbase, _CHUNK)],
                                idx_v.at[kk])
                pltpu.sync_copy(w_hbm.at[kk, pl.ds(cbase, _CHUNK)],
                                w_v.at[kk])
            handles = [pltpu.async_copy(bF_hbm.at[idx_v.at[kk]], bufs[kk], sem)
                       for kk in range(TOPK)]
            for h in handles:
                h.wait()

            def group_body(g, _):
                w16 = [w_v[kk, pl.ds(g * 16, 16)] for kk in range(TOPK)]
                for r in range(16):
                    w0 = w16[0][r]; w1 = w16[1][r]; w2 = w16[2][r]
                    row = g * 16 + r
                    for jj in range(D // 16):
                        s = pl.ds(jj * 16, 16)
                        r0_v[row, s] = (r0_v[row, s] * w0
                                        + r1_v[row, s] * w1
                                        + r2_v[row, s] * w2)
                return 0

            lax.fori_loop(0, _CHUNK // 16, group_body, 0)
            pltpu.sync_copy(r0_v, out_hbm.at[pl.ds(cbase, _CHUNK)])
            return 0

        lax.fori_loop(0, nchunks, chunk_body, 0)

    return k(w, idx, b_F)


def kernel(a_batch, a_coords, a_F, b_batch, b_coords, b_F):
    a_batch = a_batch.astype(jnp.int32)
    b_batch = b_batch.astype(jnp.int32)
    w, idx = _topk(a_batch, a_coords.astype(jnp.int32),
                   b_batch, b_coords.astype(jnp.int32))
    tmp = _gather_sum(w, idx, b_F)
    return jnp.concatenate([a_F, tmp], axis=-1)


# speedup vs baseline: 13.9696x; 1.0784x over previous
import functools

import jax
import jax.numpy as jnp
from jax import lax
from jax.experimental import pallas as pl
from jax.experimental.pallas import tpu as pltpu
from jax.experimental.pallas import tpu_sc as plsc

FULL_SCALE = 128.0
TOPK = 3
R = 0.5

BR = 256
BC = 2048
IMAX = 2**31 - 1


FMAX = float(jnp.finfo(jnp.float32).max)


def _merge33(a, b):
    x0 = jnp.minimum(a[0], b[0])
    u = jnp.maximum(a[0], b[0])
    v = jnp.minimum(a[1], b[1])
    x1 = jnp.minimum(u, v)
    x2 = jnp.minimum(jnp.maximum(u, v), jnp.minimum(a[2], b[2]))
    return (x0, x1, x2)


def _topk_kernel(a_batch_ref, b_batch_ref, a_c_ref, b_c_ref,
                 w_ref, idx_ref, keys_s):
    j = pl.program_id(1)
    ncols = pl.num_programs(1)

    @pl.when(j == 0)
    def _init():
        keys_s[...] = jnp.full((TOPK, BR, 128), FMAX, jnp.float32)

    a_b = a_batch_ref[0, :]
    b_b = b_batch_ref[0, :]
    active = jnp.logical_and(a_b[0] <= b_b[BC - 1], b_b[0] <= a_b[BR - 1])

    @pl.when(active)
    def _compute():
        a_c = (a_c_ref[...] // 16).astype(jnp.float32)
        b_c = (b_c_ref[...] // 16).astype(jnp.float32)
        ab2 = jax.lax.dot_general(a_c, b_c + b_c, (((0,), (0,)), ((), ())),
                                  preferred_element_type=jnp.float32)
        a2 = jnp.sum(a_c * a_c, axis=0)
        b2 = jnp.sum(b_c * b_c, axis=0)
        d2 = a2[:, None] + ((b2 + jnp.float32(512.0))[None, :] - ab2)
        ne = a_b[:, None] != b_b[None, :]
        d2 = jnp.where(ne, jnp.float32(66048.0), d2)
        col = jax.lax.broadcasted_iota(jnp.int32, (BR, BC), 1) + j * BC
        p = jnp.bitwise_or(jnp.left_shift(d2.astype(jnp.int32), 14), col)
        pf = jax.lax.bitcast_convert_type(p, jnp.float32)

        S = BC // 128
        sl = [pf[:, i * 128:(i + 1) * 128] for i in range(S)]
        lo = [jnp.minimum(sl[2 * i], sl[2 * i + 1]) for i in range(S // 2)]
        hi = [jnp.maximum(sl[2 * i], sl[2 * i + 1]) for i in range(S // 2)]

        def merge22(a0, a1, b0, b1):
            x0 = jnp.minimum(a0, b0)
            u = jnp.maximum(a0, b0)
            v = jnp.minimum(a1, b1)
            w = jnp.maximum(a1, b1)
            return (x0, jnp.minimum(u, v),
                    jnp.minimum(jnp.maximum(u, v), w))

        t3 = [merge22(lo[2 * i], hi[2 * i], lo[2 * i + 1], hi[2 * i + 1])
              for i in range(S // 4)]
        while len(t3) > 1:
            t3 = [_merge33(t3[2 * i], t3[2 * i + 1])
                  for i in range(len(t3) // 2)]

        run = (keys_s[0, :, :], keys_s[1, :, :], keys_s[2, :, :])
        m0, m1, m2 = _merge33(run, t3[0])
        keys_s[0, :, :] = m0
        keys_s[1, :, :] = m1
        keys_s[2, :, :] = m2

    @pl.when(j == ncols - 1)
    def _emit():
        r0 = keys_s[0, :, :]; r1 = keys_s[1, :, :]; r2 = keys_s[2, :, :]
        for k in range(TOPK):
            c = jnp.min(r0, axis=1)
            if k + 1 < TOPK:
                m = r0 == c[:, None]
                r0 = jnp.where(m, r1, r0)
                r1 = jnp.where(m, r2, r1)
                r2 = jnp.where(m, FMAX, r2)
            ci = jax.lax.bitcast_convert_type(c, jnp.int32)
            idx_ref[k, :] = jnp.bitwise_and(ci, 16383)
            d2 = (jnp.right_shift(ci, 14) - 512).astype(jnp.float32)
            dist = jnp.sqrt(d2) * jnp.float32(1.0 / FULL_SCALE)
            w_ref[k, :] = jnp.float32(R) - jnp.clip(dist, 0.0, jnp.float32(R))


def _topk(a_batch, a_coords, b_batch, b_coords):
    Na = a_batch.shape[0]
    Nb = b_batch.shape[0]
    grid = (Na // BR, Nb // BC)
    w, idx = pl.pallas_call(
        _topk_kernel,
        grid=grid,
        in_specs=[
            pl.BlockSpec((1, BR), lambda i, j: (0, i)),
            pl.BlockSpec((1, BC), lambda i, j: (0, j)),
            pl.BlockSpec((3, BR), lambda i, j: (0, i)),
            pl.BlockSpec((3, BC), lambda i, j: (0, j)),
        ],
        out_specs=[
            pl.BlockSpec((TOPK, BR), lambda i, j: (0, i)),
            pl.BlockSpec((TOPK, BR), lambda i, j: (0, i)),
        ],
        out_shape=[
            jax.ShapeDtypeStruct((TOPK, Na), jnp.float32),
            jax.ShapeDtypeStruct((TOPK, Na), jnp.int32),
        ],
        scratch_shapes=[
            pltpu.VMEM((TOPK, BR, 128), jnp.float32),
        ],
        compiler_params=pltpu.CompilerParams(
            dimension_semantics=("arbitrary", "arbitrary")),
    )(a_batch.reshape(1, Na), b_batch.reshape(1, Nb),
      a_coords.T, b_coords.T)
    return w, idx


_CHUNK = 64


def _gather_sum(w, idx, b_F):
    Na = w.shape[1]
    D = b_F.shape[1]
    info = plsc.get_sparse_core_info()
    NW = info.num_cores * info.num_subcores
    rows_per_w = Na // NW
    nchunks = rows_per_w // _CHUNK
    mesh = plsc.VectorSubcoreMesh(core_axis_name="c", subcore_axis_name="s")

    @functools.partial(
        pl.kernel, mesh=mesh,
        out_type=jax.ShapeDtypeStruct((Na, D), jnp.float32),
        scratch_types=[
            pltpu.VMEM((_CHUNK,), jnp.int32),
            pltpu.VMEM((_CHUNK,), jnp.int32),
            pltpu.VMEM((_CHUNK,), jnp.int32),
            pltpu.VMEM((_CHUNK,), jnp.float32),
            pltpu.VMEM((_CHUNK,), jnp.float32),
            pltpu.VMEM((_CHUNK,), jnp.float32),
            pltpu.VMEM((_CHUNK, D), jnp.float32),
            pltpu.VMEM((_CHUNK, D), jnp.float32),
            pltpu.VMEM((_CHUNK, D), jnp.float32),
            pltpu.SemaphoreType.DMA,
        ],
    )
    def k(w_hbm, idx_hbm, bF_hbm, out_hbm, i0_v, i1_v, i2_v,
          w0_v, w1_v, w2_v, r0_v, r1_v, r2_v, sem):
        wid = lax.axis_index("s") * info.num_cores + lax.axis_index("c")
        base = wid * rows_per_w
        bufs = (r0_v, r1_v, r2_v)
        idxs = (i0_v, i1_v, i2_v)
        ws = (w0_v, w1_v, w2_v)
        def chunk_body(c, _):
            cbase = base + c * _CHUNK
            for kk in range(TOPK):
                pltpu.sync_copy(idx_hbm.at[kk, pl.ds(cbase, _CHUNK)],
                                idxs[kk])
                pltpu.sync_copy(w_hbm.at[kk, pl.ds(cbase, _CHUNK)],
                                ws[kk])
            handles = [pltpu.async_copy(bF_hbm.at[idxs[kk]], bufs[kk], sem)
                       for kk in range(TOPK)]
            for h in handles:
                h.wait()

            def group_body(g, _):
                w16 = [ws[kk][pl.ds(g * 16, 16)] for kk in range(TOPK)]
                for r in range(16):
                    w0 = w16[0][r]; w1 = w16[1][r]; w2 = w16[2][r]
                    row = g * 16 + r
                    for jj in range(D // 16):
                        s = pl.ds(jj * 16, 16)
                        r0_v[row, s] = (r0_v[row, s] * w0
                                        + r1_v[row, s] * w1
                                        + r2_v[row, s] * w2)
                return 0

            lax.fori_loop(0, _CHUNK // 16, group_body, 0)
            pltpu.sync_copy(r0_v, out_hbm.at[pl.ds(cbase, _CHUNK)])
            return 0

        lax.fori_loop(0, nchunks, chunk_body, 0)

    return k(w, idx, b_F)


def kernel(a_batch, a_coords, a_F, b_batch, b_coords, b_F):
    a_batch = a_batch.astype(jnp.int32)
    b_batch = b_batch.astype(jnp.int32)
    a_coords = a_coords.astype(jnp.int32)
    b_coords = b_coords.astype(jnp.int32)
    Na = a_batch.shape[0]
    h = Na // 2
    tmps = []
    for lo in (0, h):
        w_h, idx_h = _topk(a_batch[lo:lo + h], a_coords[lo:lo + h],
                           b_batch, b_coords)
        tmps.append(_gather_sum(w_h, idx_h, b_F))
    return jnp.concatenate([a_F, jnp.concatenate(tmps, axis=0)], axis=-1)


# speedup vs baseline: 14.4108x; 1.0316x over previous
import functools

import jax
import jax.numpy as jnp
from jax import lax
from jax.experimental import pallas as pl
from jax.experimental.pallas import tpu as pltpu
from jax.experimental.pallas import tpu_sc as plsc

FULL_SCALE = 128.0
TOPK = 3
R = 0.5

BR = 256
BC = 2048
IMAX = 2**31 - 1


FMAX = float(jnp.finfo(jnp.float32).max)


def _merge33(a, b):
    x0 = jnp.minimum(a[0], b[0])
    u = jnp.maximum(a[0], b[0])
    v = jnp.minimum(a[1], b[1])
    x1 = jnp.minimum(u, v)
    x2 = jnp.minimum(jnp.maximum(u, v), jnp.minimum(a[2], b[2]))
    return (x0, x1, x2)


def _topk_kernel(a_batch_ref, b_batch_ref, a_c_ref, b_c_ref,
                 w_ref, idx_ref, keys_s):
    j = pl.program_id(1)
    ncols = pl.num_programs(1)

    @pl.when(j == 0)
    def _init():
        keys_s[...] = jnp.full((TOPK, BR, 128), FMAX, jnp.float32)

    a_b = a_batch_ref[0, :]
    b_b = b_batch_ref[0, :]
    active = jnp.logical_and(a_b[0] <= b_b[BC - 1], b_b[0] <= a_b[BR - 1])

    @pl.when(active)
    def _compute():
        a_c = (a_c_ref[...] // 16).astype(jnp.float32)
        b_c = (b_c_ref[...] // 16).astype(jnp.float32)
        ab2 = jax.lax.dot_general(a_c, b_c + b_c, (((0,), (0,)), ((), ())),
                                  preferred_element_type=jnp.float32)
        a2 = jnp.sum(a_c * a_c, axis=0)
        b2 = jnp.sum(b_c * b_c, axis=0)
        d2 = a2[:, None] + ((b2 + jnp.float32(512.0))[None, :] - ab2)
        ne = a_b[:, None] != b_b[None, :]
        d2 = jnp.where(ne, jnp.float32(66048.0), d2)
        col = jax.lax.broadcasted_iota(jnp.int32, (BR, BC), 1) + j * BC
        p = jnp.bitwise_or(jnp.left_shift(d2.astype(jnp.int32), 14), col)
        pf = jax.lax.bitcast_convert_type(p, jnp.float32)

        S = BC // 128
        sl = [pf[:, i * 128:(i + 1) * 128] for i in range(S)]
        lo = [jnp.minimum(sl[2 * i], sl[2 * i + 1]) for i in range(S // 2)]
        hi = [jnp.maximum(sl[2 * i], sl[2 * i + 1]) for i in range(S // 2)]

        def merge22(a0, a1, b0, b1):
            x0 = jnp.minimum(a0, b0)
            u = jnp.maximum(a0, b0)
            v = jnp.minimum(a1, b1)
            w = jnp.maximum(a1, b1)
            return (x0, jnp.minimum(u, v),
                    jnp.minimum(jnp.maximum(u, v), w))

        t3 = [merge22(lo[2 * i], hi[2 * i], lo[2 * i + 1], hi[2 * i + 1])
              for i in range(S // 4)]
        while len(t3) > 1:
            t3 = [_merge33(t3[2 * i], t3[2 * i + 1])
                  for i in range(len(t3) // 2)]

        run = (keys_s[0, :, :], keys_s[1, :, :], keys_s[2, :, :])
        m0, m1, m2 = _merge33(run, t3[0])
        keys_s[0, :, :] = m0
        keys_s[1, :, :] = m1
        keys_s[2, :, :] = m2

    @pl.when(j == ncols - 1)
    def _emit():
        r0 = keys_s[0, :, :]; r1 = keys_s[1, :, :]; r2 = keys_s[2, :, :]
        for k in range(TOPK):
            c = jnp.min(r0, axis=1)
            if k + 1 < TOPK:
                m = r0 == c[:, None]
                r0 = jnp.where(m, r1, r0)
                r1 = jnp.where(m, r2, r1)
                r2 = jnp.where(m, FMAX, r2)
            ci = jax.lax.bitcast_convert_type(c, jnp.int32)
            idx_ref[k, :] = jnp.bitwise_and(ci, 16383)
            d2 = (jnp.right_shift(ci, 14) - 512).astype(jnp.float32)
            dist = jnp.sqrt(d2) * jnp.float32(1.0 / FULL_SCALE)
            w_ref[k, :] = jnp.float32(R) - jnp.clip(dist, 0.0, jnp.float32(R))


def _topk(a_batch, a_coords, b_batch, b_coords):
    Na = a_batch.shape[0]
    Nb = b_batch.shape[0]
    grid = (Na // BR, Nb // BC)
    w, idx = pl.pallas_call(
        _topk_kernel,
        grid=grid,
        in_specs=[
            pl.BlockSpec((1, BR), lambda i, j: (0, i)),
            pl.BlockSpec((1, BC), lambda i, j: (0, j)),
            pl.BlockSpec((3, BR), lambda i, j: (0, i)),
            pl.BlockSpec((3, BC), lambda i, j: (0, j)),
        ],
        out_specs=[
            pl.BlockSpec((TOPK, BR), lambda i, j: (0, i)),
            pl.BlockSpec((TOPK, BR), lambda i, j: (0, i)),
        ],
        out_shape=[
            jax.ShapeDtypeStruct((TOPK, Na), jnp.float32),
            jax.ShapeDtypeStruct((TOPK, Na), jnp.int32),
        ],
        scratch_shapes=[
            pltpu.VMEM((TOPK, BR, 128), jnp.float32),
        ],
        compiler_params=pltpu.CompilerParams(
            dimension_semantics=("arbitrary", "arbitrary")),
    )(a_batch.reshape(1, Na), b_batch.reshape(1, Nb),
      a_coords.T, b_coords.T)
    return w, idx


_CHUNK = 64


def _gather_sum(w, idx, b_F):
    Na = w.shape[1]
    D = b_F.shape[1]
    info = plsc.get_sparse_core_info()
    NW = info.num_cores * info.num_subcores
    rows_per_w = Na // NW
    nchunks = rows_per_w // _CHUNK
    mesh = plsc.VectorSubcoreMesh(core_axis_name="c", subcore_axis_name="s")

    @functools.partial(
        pl.kernel, mesh=mesh,
        out_type=jax.ShapeDtypeStruct((Na, D), jnp.float32),
        scratch_types=[
            pltpu.VMEM((_CHUNK,), jnp.int32),
            pltpu.VMEM((_CHUNK,), jnp.int32),
            pltpu.VMEM((_CHUNK,), jnp.int32),
            pltpu.VMEM((_CHUNK,), jnp.float32),
            pltpu.VMEM((_CHUNK,), jnp.float32),
            pltpu.VMEM((_CHUNK,), jnp.float32),
            pltpu.VMEM((_CHUNK, D), jnp.float32),
            pltpu.VMEM((_CHUNK, D), jnp.float32),
            pltpu.VMEM((_CHUNK, D), jnp.float32),
            pltpu.SemaphoreType.DMA,
        ],
    )
    def k(w_hbm, idx_hbm, bF_hbm, out_hbm, i0_v, i1_v, i2_v,
          w0_v, w1_v, w2_v, r0_v, r1_v, r2_v, sem):
        wid = lax.axis_index("s") * info.num_cores + lax.axis_index("c")
        base = wid * rows_per_w
        bufs = (r0_v, r1_v, r2_v)
        idxs = (i0_v, i1_v, i2_v)
        ws = (w0_v, w1_v, w2_v)
        def chunk_body(c, _):
            cbase = base + c * _CHUNK
            for kk in range(TOPK):
                pltpu.sync_copy(idx_hbm.at[kk, pl.ds(cbase, _CHUNK)],
                                idxs[kk])
                pltpu.sync_copy(w_hbm.at[kk, pl.ds(cbase, _CHUNK)],
                                ws[kk])
            handles = [pltpu.async_copy(bF_hbm.at[idxs[kk]], bufs[kk], sem)
                       for kk in range(TOPK)]
            for h in handles:
                h.wait()

            def group_body(g, _):
                w16 = [ws[kk][pl.ds(g * 16, 16)] for kk in range(TOPK)]
                for r in range(16):
                    w0 = w16[0][r]; w1 = w16[1][r]; w2 = w16[2][r]
                    row = g * 16 + r
                    for jj in range(D // 16):
                        s = pl.ds(jj * 16, 16)
                        r0_v[row, s] = (r0_v[row, s] * w0
                                        + r1_v[row, s] * w1
                                        + r2_v[row, s] * w2)
                return 0

            lax.fori_loop(0, _CHUNK // 16, group_body, 0)
            pltpu.sync_copy(r0_v, out_hbm.at[pl.ds(cbase, _CHUNK)])
            return 0

        lax.fori_loop(0, nchunks, chunk_body, 0)

    return k(w, idx, b_F)


def kernel(a_batch, a_coords, a_F, b_batch, b_coords, b_F):
    a_batch = a_batch.astype(jnp.int32)
    b_batch = b_batch.astype(jnp.int32)
    a_coords = a_coords.astype(jnp.int32)
    b_coords = b_coords.astype(jnp.int32)
    Na = a_batch.shape[0]
    h = Na // 4
    tmps = []
    for lo in range(0, Na, h):
        w_h, idx_h = _topk(a_batch[lo:lo + h], a_coords[lo:lo + h],
                           b_batch, b_coords)
        tmps.append(_gather_sum(w_h, idx_h, b_F))
    return jnp.concatenate([a_F, jnp.concatenate(tmps, axis=0)], axis=-1)


# speedup vs baseline: 19.0412x; 1.3213x over previous
import functools

import jax
import jax.numpy as jnp
from jax import lax
from jax.experimental import pallas as pl
from jax.experimental.pallas import tpu as pltpu
from jax.experimental.pallas import tpu_sc as plsc

FULL_SCALE = 128.0
TOPK = 3
R = 0.5

BR = 512
BC = 2048
IMAX = 2**31 - 1


FMAX = float(jnp.finfo(jnp.float32).max)


def _merge33(a, b):
    x0 = jnp.minimum(a[0], b[0])
    u = jnp.maximum(a[0], b[0])
    v = jnp.minimum(a[1], b[1])
    x1 = jnp.minimum(u, v)
    x2 = jnp.minimum(jnp.maximum(u, v), jnp.minimum(a[2], b[2]))
    return (x0, x1, x2)


def _topk_kernel(a_batch_ref, b_batch_ref, a_c_ref, b_c_ref,
                 w_ref, idx_ref, keys_s):
    j = pl.program_id(1)
    ncols = pl.num_programs(1)

    @pl.when(j == 0)
    def _init():
        keys_s[...] = jnp.full((TOPK, BR, 128), FMAX, jnp.float32)

    a_b = a_batch_ref[0, :]
    b_b = b_batch_ref[0, :]
    active = jnp.logical_and(a_b[0] <= b_b[BC - 1], b_b[0] <= a_b[BR - 1])

    @pl.when(active)
    def _compute():
        a_c = (a_c_ref[...] // 16).astype(jnp.float32)
        b_c = (b_c_ref[...] // 16).astype(jnp.float32)
        ab2 = jax.lax.dot_general(a_c, b_c + b_c, (((0,), (0,)), ((), ())),
                                  preferred_element_type=jnp.float32)
        a2 = jnp.sum(a_c * a_c, axis=0)
        b2 = jnp.sum(b_c * b_c, axis=0)
        d2 = a2[:, None] + ((b2 + jnp.float32(512.0))[None, :] - ab2)
        ne = a_b[:, None] != b_b[None, :]
        d2 = jnp.where(ne, jnp.float32(66048.0), d2)
        col = jax.lax.broadcasted_iota(jnp.int32, (BR, BC), 1) + j * BC
        p = jnp.bitwise_or(jnp.left_shift(d2.astype(jnp.int32), 14), col)
        pf = jax.lax.bitcast_convert_type(p, jnp.float32)

        S = BC // 128
        sl = [pf[:, i * 128:(i + 1) * 128] for i in range(S)]
        lo = [jnp.minimum(sl[2 * i], sl[2 * i + 1]) for i in range(S // 2)]
        hi = [jnp.maximum(sl[2 * i], sl[2 * i + 1]) for i in range(S // 2)]

        def merge22(a0, a1, b0, b1):
            x0 = jnp.minimum(a0, b0)
            u = jnp.maximum(a0, b0)
            v = jnp.minimum(a1, b1)
            w = jnp.maximum(a1, b1)
            return (x0, jnp.minimum(u, v),
                    jnp.minimum(jnp.maximum(u, v), w))

        t3 = [merge22(lo[2 * i], hi[2 * i], lo[2 * i + 1], hi[2 * i + 1])
              for i in range(S // 4)]
        while len(t3) > 1:
            t3 = [_merge33(t3[2 * i], t3[2 * i + 1])
                  for i in range(len(t3) // 2)]

        run = (keys_s[0, :, :], keys_s[1, :, :], keys_s[2, :, :])
        m0, m1, m2 = _merge33(run, t3[0])
        keys_s[0, :, :] = m0
        keys_s[1, :, :] = m1
        keys_s[2, :, :] = m2

    @pl.when(j == ncols - 1)
    def _emit():
        r0 = keys_s[0, :, :]; r1 = keys_s[1, :, :]; r2 = keys_s[2, :, :]
        for k in range(TOPK):
            c = jnp.min(r0, axis=1)
            if k + 1 < TOPK:
                m = r0 == c[:, None]
                r0 = jnp.where(m, r1, r0)
                r1 = jnp.where(m, r2, r1)
                r2 = jnp.where(m, FMAX, r2)
            ci = jax.lax.bitcast_convert_type(c, jnp.int32)
            idx_ref[k, :] = jnp.bitwise_and(ci, 16383)
            d2 = (jnp.right_shift(ci, 14) - 512).astype(jnp.float32)
            dist = jnp.sqrt(d2) * jnp.float32(1.0 / FULL_SCALE)
            w_ref[k, :] = jnp.float32(R) - jnp.clip(dist, 0.0, jnp.float32(R))


def _topk(a_batch, a_coords, b_batch, b_coords):
    Na = a_batch.shape[0]
    Nb = b_batch.shape[0]
    grid = (Na // BR, Nb // BC)
    w, idx = pl.pallas_call(
        _topk_kernel,
        grid=grid,
        in_specs=[
            pl.BlockSpec((1, BR), lambda i, j: (0, i)),
            pl.BlockSpec((1, BC), lambda i, j: (0, j)),
            pl.BlockSpec((3, BR), lambda i, j: (0, i)),
            pl.BlockSpec((3, BC), lambda i, j: (0, j)),
        ],
        out_specs=[
            pl.BlockSpec((TOPK, BR), lambda i, j: (0, i)),
            pl.BlockSpec((TOPK, BR), lambda i, j: (0, i)),
        ],
        out_shape=[
            jax.ShapeDtypeStruct((TOPK, Na), jnp.float32),
            jax.ShapeDtypeStruct((TOPK, Na), jnp.int32),
        ],
        scratch_shapes=[
            pltpu.VMEM((TOPK, BR, 128), jnp.float32),
        ],
        compiler_params=pltpu.CompilerParams(
            dimension_semantics=("arbitrary", "arbitrary")),
    )(a_batch.reshape(1, Na), b_batch.reshape(1, Nb),
      a_coords.T, b_coords.T)
    return w, idx


_CHUNK = 64


def _gather_sum(w, idx, b_F):
    Na = w.shape[1]
    D = b_F.shape[1]
    info = plsc.get_sparse_core_info()
    NW = info.num_cores * info.num_subcores
    rows_per_w = Na // NW
    nchunks = rows_per_w // _CHUNK
    mesh = plsc.VectorSubcoreMesh(core_axis_name="c", subcore_axis_name="s")

    @functools.partial(
        pl.kernel, mesh=mesh,
        out_type=jax.ShapeDtypeStruct((Na, D), jnp.float32),
        scratch_types=[
            pltpu.VMEM((_CHUNK,), jnp.int32),
            pltpu.VMEM((_CHUNK,), jnp.int32),
            pltpu.VMEM((_CHUNK,), jnp.int32),
            pltpu.VMEM((_CHUNK,), jnp.float32),
            pltpu.VMEM((_CHUNK,), jnp.float32),
            pltpu.VMEM((_CHUNK,), jnp.float32),
            pltpu.VMEM((_CHUNK, D), jnp.float32),
            pltpu.VMEM((_CHUNK, D), jnp.float32),
            pltpu.VMEM((_CHUNK, D), jnp.float32),
            pltpu.SemaphoreType.DMA,
        ],
    )
    def k(w_hbm, idx_hbm, bF_hbm, out_hbm, i0_v, i1_v, i2_v,
          w0_v, w1_v, w2_v, r0_v, r1_v, r2_v, sem):
        wid = lax.axis_index("s") * info.num_cores + lax.axis_index("c")
        base = wid * rows_per_w
        bufs = (r0_v, r1_v, r2_v)
        idxs = (i0_v, i1_v, i2_v)
        ws = (w0_v, w1_v, w2_v)
        def chunk_body(c, _):
            cbase = base + c * _CHUNK
            for kk in range(TOPK):
                pltpu.sync_copy(idx_hbm.at[kk, pl.ds(cbase, _CHUNK)],
                                idxs[kk])
                pltpu.sync_copy(w_hbm.at[kk, pl.ds(cbase, _CHUNK)],
                                ws[kk])
            handles = [pltpu.async_copy(bF_hbm.at[idxs[kk]], bufs[kk], sem)
                       for kk in range(TOPK)]
            for h in handles:
                h.wait()

            def group_body(g, _):
                w16 = [ws[kk][pl.ds(g * 16, 16)] for kk in range(TOPK)]
                for r in range(16):
                    w0 = w16[0][r]; w1 = w16[1][r]; w2 = w16[2][r]
                    row = g * 16 + r
                    for jj in range(D // 16):
                        s = pl.ds(jj * 16, 16)
                        r0_v[row, s] = (r0_v[row, s] * w0
                                        + r1_v[row, s] * w1
                                        + r2_v[row, s] * w2)
                return 0

            lax.fori_loop(0, _CHUNK // 16, group_body, 0)
            pltpu.sync_copy(r0_v, out_hbm.at[pl.ds(cbase, _CHUNK)])
            return 0

        lax.fori_loop(0, nchunks, chunk_body, 0)

    return k(w, idx, b_F)


def kernel(a_batch, a_coords, a_F, b_batch, b_coords, b_F):
    a_batch = a_batch.astype(jnp.int32)
    b_batch = b_batch.astype(jnp.int32)
    a_coords = a_coords.astype(jnp.int32)
    b_coords = b_coords.astype(jnp.int32)
    Na = a_batch.shape[0]
    h = Na // 4
    tmps = []
    for lo in range(0, Na, h):
        w_h, idx_h = _topk(a_batch[lo:lo + h], a_coords[lo:lo + h],
                           b_batch, b_coords)
        tmps.append(_gather_sum(w_h, idx_h, b_F))
    return jnp.concatenate([a_F, jnp.concatenate(tmps, axis=0)], axis=-1)


# speedup vs baseline: 22.1512x; 1.1633x over previous
import functools

import jax
import jax.numpy as jnp
from jax import lax
from jax.experimental import pallas as pl
from jax.experimental.pallas import tpu as pltpu
from jax.experimental.pallas import tpu_sc as plsc

FULL_SCALE = 128.0
TOPK = 3
R = 0.5

BR = 1024
BC = 2048
IMAX = 2**31 - 1


FMAX = float(jnp.finfo(jnp.float32).max)


def _merge33(a, b):
    x0 = jnp.minimum(a[0], b[0])
    u = jnp.maximum(a[0], b[0])
    v = jnp.minimum(a[1], b[1])
    x1 = jnp.minimum(u, v)
    x2 = jnp.minimum(jnp.maximum(u, v), jnp.minimum(a[2], b[2]))
    return (x0, x1, x2)


def _topk_kernel(a_batch_ref, b_batch_ref, a_c_ref, b_c_ref,
                 w_ref, idx_ref, keys_s):
    j = pl.program_id(1)
    ncols = pl.num_programs(1)

    @pl.when(j == 0)
    def _init():
        keys_s[...] = jnp.full((TOPK, BR, 128), FMAX, jnp.float32)

    a_b = a_batch_ref[0, :]
    b_b = b_batch_ref[0, :]
    active = jnp.logical_and(a_b[0] <= b_b[BC - 1], b_b[0] <= a_b[BR - 1])

    @pl.when(active)
    def _compute():
        a_c = (a_c_ref[...] // 16).astype(jnp.float32)
        b_c = (b_c_ref[...] // 16).astype(jnp.float32)
        ab2 = jax.lax.dot_general(a_c, b_c + b_c, (((0,), (0,)), ((), ())),
                                  preferred_element_type=jnp.float32)
        a2 = jnp.sum(a_c * a_c, axis=0)
        b2 = jnp.sum(b_c * b_c, axis=0)
        d2 = a2[:, None] + ((b2 + jnp.float32(512.0))[None, :] - ab2)
        ne = a_b[:, None] != b_b[None, :]
        d2 = jnp.where(ne, jnp.float32(66048.0), d2)
        col = jax.lax.broadcasted_iota(jnp.int32, (BR, BC), 1) + j * BC
        p = jnp.bitwise_or(jnp.left_shift(d2.astype(jnp.int32), 14), col)
        pf = jax.lax.bitcast_convert_type(p, jnp.float32)

        S = BC // 128
        sl = [pf[:, i * 128:(i + 1) * 128] for i in range(S)]
        lo = [jnp.minimum(sl[2 * i], sl[2 * i + 1]) for i in range(S // 2)]
        hi = [jnp.maximum(sl[2 * i], sl[2 * i + 1]) for i in range(S // 2)]

        def merge22(a0, a1, b0, b1):
            x0 = jnp.minimum(a0, b0)
            u = jnp.maximum(a0, b0)
            v = jnp.minimum(a1, b1)
            w = jnp.maximum(a1, b1)
            return (x0, jnp.minimum(u, v),
                    jnp.minimum(jnp.maximum(u, v), w))

        t3 = [merge22(lo[2 * i], hi[2 * i], lo[2 * i + 1], hi[2 * i + 1])
              for i in range(S // 4)]
        while len(t3) > 1:
            t3 = [_merge33(t3[2 * i], t3[2 * i + 1])
                  for i in range(len(t3) // 2)]

        run = (keys_s[0, :, :], keys_s[1, :, :], keys_s[2, :, :])
        m0, m1, m2 = _merge33(run, t3[0])
        keys_s[0, :, :] = m0
        keys_s[1, :, :] = m1
        keys_s[2, :, :] = m2

    @pl.when(j == ncols - 1)
    def _emit():
        r0 = keys_s[0, :, :]; r1 = keys_s[1, :, :]; r2 = keys_s[2, :, :]
        for k in range(TOPK):
            c = jnp.min(r0, axis=1)
            if k + 1 < TOPK:
                m = r0 == c[:, None]
                r0 = jnp.where(m, r1, r0)
                r1 = jnp.where(m, r2, r1)
                r2 = jnp.where(m, FMAX, r2)
            ci = jax.lax.bitcast_convert_type(c, jnp.int32)
            idx_ref[k, :] = jnp.bitwise_and(ci, 16383)
            d2 = (jnp.right_shift(ci, 14) - 512).astype(jnp.float32)
            dist = jnp.sqrt(d2) * jnp.float32(1.0 / FULL_SCALE)
            w_ref[k, :] = jnp.float32(R) - jnp.clip(dist, 0.0, jnp.float32(R))


def _topk(a_batch, a_coords, b_batch, b_coords):
    Na = a_batch.shape[0]
    Nb = b_batch.shape[0]
    grid = (Na // BR, Nb // BC)
    w, idx = pl.pallas_call(
        _topk_kernel,
        grid=grid,
        in_specs=[
            pl.BlockSpec((1, BR), lambda i, j: (0, i)),
            pl.BlockSpec((1, BC), lambda i, j: (0, j)),
            pl.BlockSpec((3, BR), lambda i, j: (0, i)),
            pl.BlockSpec((3, BC), lambda i, j: (0, j)),
        ],
        out_specs=[
            pl.BlockSpec((TOPK, BR), lambda i, j: (0, i)),
            pl.BlockSpec((TOPK, BR), lambda i, j: (0, i)),
        ],
        out_shape=[
            jax.ShapeDtypeStruct((TOPK, Na), jnp.float32),
            jax.ShapeDtypeStruct((TOPK, Na), jnp.int32),
        ],
        scratch_shapes=[
            pltpu.VMEM((TOPK, BR, 128), jnp.float32),
        ],
        compiler_params=pltpu.CompilerParams(
            dimension_semantics=("arbitrary", "arbitrary")),
    )(a_batch.reshape(1, Na), b_batch.reshape(1, Nb),
      a_coords.T, b_coords.T)
    return w, idx


_CHUNK = 64


def _gather_sum(w, idx, b_F):
    Na = w.shape[1]
    D = b_F.shape[1]
    info = plsc.get_sparse_core_info()
    NW = info.num_cores * info.num_subcores
    rows_per_w = Na // NW
    nchunks = rows_per_w // _CHUNK
    mesh = plsc.VectorSubcoreMesh(core_axis_name="c", subcore_axis_name="s")

    @functools.partial(
        pl.kernel, mesh=mesh,
        out_type=jax.ShapeDtypeStruct((Na, D), jnp.float32),
        scratch_types=[
            pltpu.VMEM((_CHUNK,), jnp.int32),
            pltpu.VMEM((_CHUNK,), jnp.int32),
            pltpu.VMEM((_CHUNK,), jnp.int32),
            pltpu.VMEM((_CHUNK,), jnp.float32),
            pltpu.VMEM((_CHUNK,), jnp.float32),
            pltpu.VMEM((_CHUNK,), jnp.float32),
            pltpu.VMEM((_CHUNK, D), jnp.float32),
            pltpu.VMEM((_CHUNK, D), jnp.float32),
            pltpu.VMEM((_CHUNK, D), jnp.float32),
            pltpu.SemaphoreType.DMA,
        ],
    )
    def k(w_hbm, idx_hbm, bF_hbm, out_hbm, i0_v, i1_v, i2_v,
          w0_v, w1_v, w2_v, r0_v, r1_v, r2_v, sem):
        wid = lax.axis_index("s") * info.num_cores + lax.axis_index("c")
        base = wid * rows_per_w
        bufs = (r0_v, r1_v, r2_v)
        idxs = (i0_v, i1_v, i2_v)
        ws = (w0_v, w1_v, w2_v)
        def chunk_body(c, _):
            cbase = base + c * _CHUNK
            for kk in range(TOPK):
                pltpu.sync_copy(idx_hbm.at[kk, pl.ds(cbase, _CHUNK)],
                                idxs[kk])
                pltpu.sync_copy(w_hbm.at[kk, pl.ds(cbase, _CHUNK)],
                                ws[kk])
            handles = [pltpu.async_copy(bF_hbm.at[idxs[kk]], bufs[kk], sem)
                       for kk in range(TOPK)]
            for h in handles:
                h.wait()

            def group_body(g, _):
                w16 = [ws[kk][pl.ds(g * 16, 16)] for kk in range(TOPK)]
                for r in range(16):
                    w0 = w16[0][r]; w1 = w16[1][r]; w2 = w16[2][r]
                    row = g * 16 + r
                    for jj in range(D // 16):
                        s = pl.ds(jj * 16, 16)
                        r0_v[row, s] = (r0_v[row, s] * w0
                                        + r1_v[row, s] * w1
                                        + r2_v[row, s] * w2)
                return 0

            lax.fori_loop(0, _CHUNK // 16, group_body, 0)
            pltpu.sync_copy(r0_v, out_hbm.at[pl.ds(cbase, _CHUNK)])
            return 0

        lax.fori_loop(0, nchunks, chunk_body, 0)

    return k(w, idx, b_F)


def kernel(a_batch, a_coords, a_F, b_batch, b_coords, b_F):
    a_batch = a_batch.astype(jnp.int32)
    b_batch = b_batch.astype(jnp.int32)
    a_coords = a_coords.astype(jnp.int32)
    b_coords = b_coords.astype(jnp.int32)
    Na = a_batch.shape[0]
    h = Na // 4
    tmps = []
    for lo in range(0, Na, h):
        w_h, idx_h = _topk(a_batch[lo:lo + h], a_coords[lo:lo + h],
                           b_batch, b_coords)
        tmps.append(_gather_sum(w_h, idx_h, b_F))
    return jnp.concatenate([a_F, jnp.concatenate(tmps, axis=0)], axis=-1)


# speedup vs baseline: 22.8168x; 1.0300x over previous
import functools

import jax
import jax.numpy as jnp
from jax import lax
from jax.experimental import pallas as pl
from jax.experimental.pallas import tpu as pltpu
from jax.experimental.pallas import tpu_sc as plsc

FULL_SCALE = 128.0
TOPK = 3
R = 0.5

BR = 2048
BC = 2048
IMAX = 2**31 - 1


FMAX = float(jnp.finfo(jnp.float32).max)


def _merge33(a, b):
    x0 = jnp.minimum(a[0], b[0])
    u = jnp.maximum(a[0], b[0])
    v = jnp.minimum(a[1], b[1])
    x1 = jnp.minimum(u, v)
    x2 = jnp.minimum(jnp.maximum(u, v), jnp.minimum(a[2], b[2]))
    return (x0, x1, x2)


def _topk_kernel(a_batch_ref, b_batch_ref, a_c_ref, b_c_ref,
                 w_ref, idx_ref, keys_s):
    j = pl.program_id(1)
    ncols = pl.num_programs(1)

    @pl.when(j == 0)
    def _init():
        keys_s[...] = jnp.full((TOPK, BR, 128), FMAX, jnp.float32)

    a_b = a_batch_ref[0, :]
    b_b = b_batch_ref[0, :]
    active = jnp.logical_and(a_b[0] <= b_b[BC - 1], b_b[0] <= a_b[BR - 1])

    @pl.when(active)
    def _compute():
        a_c = (a_c_ref[...] // 16).astype(jnp.float32)
        b_c = (b_c_ref[...] // 16).astype(jnp.float32)
        ab2 = jax.lax.dot_general(a_c, b_c + b_c, (((0,), (0,)), ((), ())),
                                  preferred_element_type=jnp.float32)
        a2 = jnp.sum(a_c * a_c, axis=0)
        b2 = jnp.sum(b_c * b_c, axis=0)
        d2 = a2[:, None] + ((b2 + jnp.float32(512.0))[None, :] - ab2)
        ne = a_b[:, None] != b_b[None, :]
        d2 = jnp.where(ne, jnp.float32(66048.0), d2)
        col = jax.lax.broadcasted_iota(jnp.int32, (BR, BC), 1) + j * BC
        p = jnp.bitwise_or(jnp.left_shift(d2.astype(jnp.int32), 14), col)
        pf = jax.lax.bitcast_convert_type(p, jnp.float32)

        S = BC // 128
        sl = [pf[:, i * 128:(i + 1) * 128] for i in range(S)]
        lo = [jnp.minimum(sl[2 * i], sl[2 * i + 1]) for i in range(S // 2)]
        hi = [jnp.maximum(sl[2 * i], sl[2 * i + 1]) for i in range(S // 2)]

        def merge22(a0, a1, b0, b1):
            x0 = jnp.minimum(a0, b0)
            u = jnp.maximum(a0, b0)
            v = jnp.minimum(a1, b1)
            w = jnp.maximum(a1, b1)
            return (x0, jnp.minimum(u, v),
                    jnp.minimum(jnp.maximum(u, v), w))

        t3 = [merge22(lo[2 * i], hi[2 * i], lo[2 * i + 1], hi[2 * i + 1])
              for i in range(S // 4)]
        while len(t3) > 1:
            t3 = [_merge33(t3[2 * i], t3[2 * i + 1])
                  for i in range(len(t3) // 2)]

        run = (keys_s[0, :, :], keys_s[1, :, :], keys_s[2, :, :])
        m0, m1, m2 = _merge33(run, t3[0])
        keys_s[0, :, :] = m0
        keys_s[1, :, :] = m1
        keys_s[2, :, :] = m2

    @pl.when(j == ncols - 1)
    def _emit():
        r0 = keys_s[0, :, :]; r1 = keys_s[1, :, :]; r2 = keys_s[2, :, :]
        for k in range(TOPK):
            c = jnp.min(r0, axis=1)
            if k + 1 < TOPK:
                m = r0 == c[:, None]
                r0 = jnp.where(m, r1, r0)
                r1 = jnp.where(m, r2, r1)
                r2 = jnp.where(m, FMAX, r2)
            ci = jax.lax.bitcast_convert_type(c, jnp.int32)
            idx_ref[k, :] = jnp.bitwise_and(ci, 16383)
            d2 = (jnp.right_shift(ci, 14) - 512).astype(jnp.float32)
            dist = jnp.sqrt(d2) * jnp.float32(1.0 / FULL_SCALE)
            w_ref[k, :] = jnp.float32(R) - jnp.clip(dist, 0.0, jnp.float32(R))


def _topk(a_batch, a_coords, b_batch, b_coords):
    Na = a_batch.shape[0]
    Nb = b_batch.shape[0]
    grid = (Na // BR, Nb // BC)
    w, idx = pl.pallas_call(
        _topk_kernel,
        grid=grid,
        in_specs=[
            pl.BlockSpec((1, BR), lambda i, j: (0, i)),
            pl.BlockSpec((1, BC), lambda i, j: (0, j)),
            pl.BlockSpec((3, BR), lambda i, j: (0, i)),
            pl.BlockSpec((3, BC), lambda i, j: (0, j)),
        ],
        out_specs=[
            pl.BlockSpec((TOPK, BR), lambda i, j: (0, i)),
            pl.BlockSpec((TOPK, BR), lambda i, j: (0, i)),
        ],
        out_shape=[
            jax.ShapeDtypeStruct((TOPK, Na), jnp.float32),
            jax.ShapeDtypeStruct((TOPK, Na), jnp.int32),
        ],
        scratch_shapes=[
            pltpu.VMEM((TOPK, BR, 128), jnp.float32),
        ],
        compiler_params=pltpu.CompilerParams(
            dimension_semantics=("arbitrary", "arbitrary")),
    )(a_batch.reshape(1, Na), b_batch.reshape(1, Nb),
      a_coords.T, b_coords.T)
    return w, idx


_CHUNK = 64


def _gather_sum(w, idx, b_F):
    Na = w.shape[1]
    D = b_F.shape[1]
    info = plsc.get_sparse_core_info()
    NW = info.num_cores * info.num_subcores
    rows_per_w = Na // NW
    nchunks = rows_per_w // _CHUNK
    mesh = plsc.VectorSubcoreMesh(core_axis_name="c", subcore_axis_name="s")

    @functools.partial(
        pl.kernel, mesh=mesh,
        out_type=jax.ShapeDtypeStruct((Na, D), jnp.float32),
        scratch_types=[
            pltpu.VMEM((_CHUNK,), jnp.int32),
            pltpu.VMEM((_CHUNK,), jnp.int32),
            pltpu.VMEM((_CHUNK,), jnp.int32),
            pltpu.VMEM((_CHUNK,), jnp.float32),
            pltpu.VMEM((_CHUNK,), jnp.float32),
            pltpu.VMEM((_CHUNK,), jnp.float32),
            pltpu.VMEM((_CHUNK, D), jnp.float32),
            pltpu.VMEM((_CHUNK, D), jnp.float32),
            pltpu.VMEM((_CHUNK, D), jnp.float32),
            pltpu.SemaphoreType.DMA,
        ],
    )
    def k(w_hbm, idx_hbm, bF_hbm, out_hbm, i0_v, i1_v, i2_v,
          w0_v, w1_v, w2_v, r0_v, r1_v, r2_v, sem):
        wid = lax.axis_index("s") * info.num_cores + lax.axis_index("c")
        base = wid * rows_per_w
        bufs = (r0_v, r1_v, r2_v)
        idxs = (i0_v, i1_v, i2_v)
        ws = (w0_v, w1_v, w2_v)
        def chunk_body(c, _):
            cbase = base + c * _CHUNK
            for kk in range(TOPK):
                pltpu.sync_copy(idx_hbm.at[kk, pl.ds(cbase, _CHUNK)],
                                idxs[kk])
                pltpu.sync_copy(w_hbm.at[kk, pl.ds(cbase, _CHUNK)],
                                ws[kk])
            handles = [pltpu.async_copy(bF_hbm.at[idxs[kk]], bufs[kk], sem)
                       for kk in range(TOPK)]
            for h in handles:
                h.wait()

            def group_body(g, _):
                w16 = [ws[kk][pl.ds(g * 16, 16)] for kk in range(TOPK)]
                for r in range(16):
                    w0 = w16[0][r]; w1 = w16[1][r]; w2 = w16[2][r]
                    row = g * 16 + r
                    for jj in range(D // 16):
                        s = pl.ds(jj * 16, 16)
                        r0_v[row, s] = (r0_v[row, s] * w0
                                        + r1_v[row, s] * w1
                                        + r2_v[row, s] * w2)
                return 0

            lax.fori_loop(0, _CHUNK // 16, group_body, 0)
            pltpu.sync_copy(r0_v, out_hbm.at[pl.ds(cbase, _CHUNK)])
            return 0

        lax.fori_loop(0, nchunks, chunk_body, 0)

    return k(w, idx, b_F)


def kernel(a_batch, a_coords, a_F, b_batch, b_coords, b_F):
    a_batch = a_batch.astype(jnp.int32)
    b_batch = b_batch.astype(jnp.int32)
    a_coords = a_coords.astype(jnp.int32)
    b_coords = b_coords.astype(jnp.int32)
    Na = a_batch.shape[0]
    h = Na // 4
    tmps = []
    for lo in range(0, Na, h):
        w_h, idx_h = _topk(a_batch[lo:lo + h], a_coords[lo:lo + h],
                           b_batch, b_coords)
        tmps.append(_gather_sum(w_h, idx_h, b_F))
    return jnp.concatenate([a_F, jnp.concatenate(tmps, axis=0)], axis=-1)


# speedup vs baseline: 23.5114x; 1.0304x over previous
import functools

import jax
import jax.numpy as jnp
from jax import lax
from jax.experimental import pallas as pl
from jax.experimental.pallas import tpu as pltpu
from jax.experimental.pallas import tpu_sc as plsc

FULL_SCALE = 128.0
TOPK = 3
R = 0.5

BR = 2048
BC = 2048
IMAX = 2**31 - 1


FMAX = float(jnp.finfo(jnp.float32).max)


def _merge33(a, b):
    x0 = jnp.minimum(a[0], b[0])
    u = jnp.maximum(a[0], b[0])
    v = jnp.minimum(a[1], b[1])
    x1 = jnp.minimum(u, v)
    x2 = jnp.minimum(jnp.maximum(u, v), jnp.minimum(a[2], b[2]))
    return (x0, x1, x2)


def _topk_kernel(a_batch_ref, b_batch_ref, a_c_ref, b_c_ref,
                 w_ref, idx_ref, keys_s):
    j = pl.program_id(1)
    ncols = pl.num_programs(1)

    @pl.when(j == 0)
    def _init():
        keys_s[...] = jnp.full((TOPK, BR, 128), FMAX, jnp.float32)

    a_b = a_batch_ref[0, :]
    b_b = b_batch_ref[0, :]
    active = jnp.logical_and(a_b[0] <= b_b[BC - 1], b_b[0] <= a_b[BR - 1])

    @pl.when(active)
    def _compute():
        a_c = a_c_ref[...]
        b_c = b_c_ref[...]
        ab2 = jax.lax.dot_general(a_c, b_c + b_c, (((0,), (0,)), ((), ())),
                                  preferred_element_type=jnp.float32)
        a2 = jnp.sum(a_c * a_c, axis=0)
        b2 = jnp.sum(b_c * b_c, axis=0)
        d2 = a2[:, None] + ((b2 + jnp.float32(512.0))[None, :] - ab2)
        d2 = jnp.minimum(d2, jnp.float32(66048.0))
        col = jax.lax.broadcasted_iota(jnp.int32, (BR, BC), 1) + j * BC
        p = jnp.bitwise_or(jnp.left_shift(d2.astype(jnp.int32), 14), col)
        pf = jax.lax.bitcast_convert_type(p, jnp.float32)

        S = BC // 128
        sl = [pf[:, i * 128:(i + 1) * 128] for i in range(S)]
        lo = [jnp.minimum(sl[2 * i], sl[2 * i + 1]) for i in range(S // 2)]
        hi = [jnp.maximum(sl[2 * i], sl[2 * i + 1]) for i in range(S // 2)]

        def merge22(a0, a1, b0, b1):
            x0 = jnp.minimum(a0, b0)
            u = jnp.maximum(a0, b0)
            v = jnp.minimum(a1, b1)
            w = jnp.maximum(a1, b1)
            return (x0, jnp.minimum(u, v),
                    jnp.minimum(jnp.maximum(u, v), w))

        t3 = [merge22(lo[2 * i], hi[2 * i], lo[2 * i + 1], hi[2 * i + 1])
              for i in range(S // 4)]
        while len(t3) > 1:
            t3 = [_merge33(t3[2 * i], t3[2 * i + 1])
                  for i in range(len(t3) // 2)]

        run = (keys_s[0, :, :], keys_s[1, :, :], keys_s[2, :, :])
        m0, m1, m2 = _merge33(run, t3[0])
        keys_s[0, :, :] = m0
        keys_s[1, :, :] = m1
        keys_s[2, :, :] = m2

    @pl.when(j == ncols - 1)
    def _emit():
        r0 = keys_s[0, :, :]; r1 = keys_s[1, :, :]; r2 = keys_s[2, :, :]
        for k in range(TOPK):
            c = jnp.min(r0, axis=1)
            if k + 1 < TOPK:
                m = r0 == c[:, None]
                r0 = jnp.where(m, r1, r0)
                r1 = jnp.where(m, r2, r1)
                r2 = jnp.where(m, FMAX, r2)
            ci = jax.lax.bitcast_convert_type(c, jnp.int32)
            idx_ref[k, :] = jnp.bitwise_and(ci, 16383)
            d2 = (jnp.right_shift(ci, 14) - 512).astype(jnp.float32)
            dist = jnp.sqrt(d2) * jnp.float32(1.0 / FULL_SCALE)
            w_ref[k, :] = jnp.float32(R) - jnp.clip(dist, 0.0, jnp.float32(R))


def _topk(a_batch, a_coords, b_batch, b_coords):
    Na = a_batch.shape[0]
    Nb = b_batch.shape[0]
    grid = (Na // BR, Nb // BC)
    w, idx = pl.pallas_call(
        _topk_kernel,
        grid=grid,
        in_specs=[
            pl.BlockSpec((1, BR), lambda i, j: (0, i)),
            pl.BlockSpec((1, BC), lambda i, j: (0, j)),
            pl.BlockSpec((4, BR), lambda i, j: (0, i)),
            pl.BlockSpec((4, BC), lambda i, j: (0, j)),
        ],
        out_specs=[
            pl.BlockSpec((TOPK, BR), lambda i, j: (0, i)),
            pl.BlockSpec((TOPK, BR), lambda i, j: (0, i)),
        ],
        out_shape=[
            jax.ShapeDtypeStruct((TOPK, Na), jnp.float32),
            jax.ShapeDtypeStruct((TOPK, Na), jnp.int32),
        ],
        scratch_shapes=[
            pltpu.VMEM((TOPK, BR, 128), jnp.float32),
        ],
        compiler_params=pltpu.CompilerParams(
            dimension_semantics=("arbitrary", "arbitrary")),
    )(a_batch.reshape(1, Na), b_batch.reshape(1, Nb),
      a_coords, b_coords)
    return w, idx


_CHUNK = 64


def _gather_sum(w, idx, b_F):
    Na = w.shape[1]
    D = b_F.shape[1]
    info = plsc.get_sparse_core_info()
    NW = info.num_cores * info.num_subcores
    rows_per_w = Na // NW
    nchunks = rows_per_w // _CHUNK
    mesh = plsc.VectorSubcoreMesh(core_axis_name="c", subcore_axis_name="s")

    @functools.partial(
        pl.kernel, mesh=mesh,
        out_type=jax.ShapeDtypeStruct((Na, D), jnp.float32),
        scratch_types=[
            pltpu.VMEM((_CHUNK,), jnp.int32),
            pltpu.VMEM((_CHUNK,), jnp.int32),
            pltpu.VMEM((_CHUNK,), jnp.int32),
            pltpu.VMEM((_CHUNK,), jnp.float32),
            pltpu.VMEM((_CHUNK,), jnp.float32),
            pltpu.VMEM((_CHUNK,), jnp.float32),
            pltpu.VMEM((_CHUNK, D), jnp.float32),
            pltpu.VMEM((_CHUNK, D), jnp.float32),
            pltpu.VMEM((_CHUNK, D), jnp.float32),
            pltpu.SemaphoreType.DMA,
        ],
    )
    def k(w_hbm, idx_hbm, bF_hbm, out_hbm, i0_v, i1_v, i2_v,
          w0_v, w1_v, w2_v, r0_v, r1_v, r2_v, sem):
        wid = lax.axis_index("s") * info.num_cores + lax.axis_index("c")
        base = wid * rows_per_w
        bufs = (r0_v, r1_v, r2_v)
        idxs = (i0_v, i1_v, i2_v)
        ws = (w0_v, w1_v, w2_v)
        def chunk_body(c, _):
            cbase = base + c * _CHUNK
            for kk in range(TOPK):
                pltpu.sync_copy(idx_hbm.at[kk, pl.ds(cbase, _CHUNK)],
                                idxs[kk])
                pltpu.sync_copy(w_hbm.at[kk, pl.ds(cbase, _CHUNK)],
                                ws[kk])
            handles = [pltpu.async_copy(bF_hbm.at[idxs[kk]], bufs[kk], sem)
                       for kk in range(TOPK)]
            for h in handles:
                h.wait()

            def group_body(g, _):
                w16 = [ws[kk][pl.ds(g * 16, 16)] for kk in range(TOPK)]
                for r in range(16):
                    w0 = w16[0][r]; w1 = w16[1][r]; w2 = w16[2][r]
                    row = g * 16 + r
                    for jj in range(D // 16):
                        s = pl.ds(jj * 16, 16)
                        r0_v[row, s] = (r0_v[row, s] * w0
                                        + r1_v[row, s] * w1
                                        + r2_v[row, s] * w2)
                return 0

            lax.fori_loop(0, _CHUNK // 16, group_body, 0)
            pltpu.sync_copy(r0_v, out_hbm.at[pl.ds(cbase, _CHUNK)])
            return 0

        lax.fori_loop(0, nchunks, chunk_body, 0)

    return k(w, idx, b_F)


def kernel(a_batch, a_coords, a_F, b_batch, b_coords, b_F):
    a_batch = a_batch.astype(jnp.int32)
    b_batch = b_batch.astype(jnp.int32)
    a_c4 = jnp.concatenate(
        [(a_coords.astype(jnp.int32) // 16).T,
         a_batch[None, :] * 256], axis=0).astype(jnp.float32)
    b_c4 = jnp.concatenate(
        [(b_coords.astype(jnp.int32) // 16).T,
         b_batch[None, :] * 256], axis=0).astype(jnp.float32)
    Na = a_batch.shape[0]
    h = Na // 4
    tmps = []
    for lo in range(0, Na, h):
        w_h, idx_h = _topk(a_batch[lo:lo + h], a_c4[:, lo:lo + h],
                           b_batch, b_c4)
        tmps.append(_gather_sum(w_h, idx_h, b_F))
    return jnp.concatenate([a_F, jnp.concatenate(tmps, axis=0)], axis=-1)
